# baseline (speedup 1.0000x reference)
import jax
import jax.numpy as jnp
from jax.experimental import pallas as pl
from jax.experimental.pallas import tpu as pltpu

VIS_LEN = 50
IMU_LEN = 200
SEQ = VIS_LEN + IMU_LEN
HIDDEN = 128
EPS = 1e-12
BBLK = 16


def _body(vis_ref, aud_ref, pvt_ref, pit_ref, mt_ref, e1_ref, e2_ref,
          g_ref, b_ref, out_ref):
    m0 = mt_ref[0:1, :]
    m1 = mt_ref[1:2, :]
    bias_vis = pvt_ref[...] + m0
    bias_imu = pit_ref[...] + m1

    vis = vis_ref[...]
    aud = aud_ref[...]
    n = vis.shape[0]

    x0 = jnp.broadcast_to((e1_ref[...] + bias_vis[0:1, :])[None], (n, 1, HIDDEN))
    xv = vis + bias_vis[1:, :][None]
    x1 = jnp.broadcast_to((e2_ref[...] + bias_imu[0:1, :])[None], (n, 1, HIDDEN))
    xa = aud + bias_imu[1:, :][None]
    x = jnp.concatenate([x0, xv, x1, xa], axis=1)

    s1 = jnp.sum(x, axis=-1, keepdims=True)
    s2 = jnp.sum(x * x, axis=-1, keepdims=True)
    mu = s1 * (1.0 / HIDDEN)
    var = s2 * (1.0 / HIDDEN) - mu * mu
    r = jax.lax.rsqrt(var + EPS)
    out_ref[...] = (x - mu) * r * g_ref[...] + b_ref[...]


def kernel(visual_embedding, audio_embedding, posi_visual_table,
           posi_imu_table, modal_table, esp_1, esp_2, ln_gamma, ln_beta):
    B = visual_embedding.shape[0]
    grid = (B // BBLK,)
    out = pl.pallas_call(
        _body,
        grid=grid,
        in_specs=[
            pl.BlockSpec((BBLK, VIS_LEN - 1, HIDDEN), lambda i: (i, 0, 0)),
            pl.BlockSpec((BBLK, IMU_LEN - 1, HIDDEN), lambda i: (i, 0, 0)),
            pl.BlockSpec((VIS_LEN, HIDDEN), lambda i: (0, 0)),
            pl.BlockSpec((IMU_LEN, HIDDEN), lambda i: (0, 0)),
            pl.BlockSpec((2, HIDDEN), lambda i: (0, 0)),
            pl.BlockSpec((1, HIDDEN), lambda i: (0, 0)),
            pl.BlockSpec((1, HIDDEN), lambda i: (0, 0)),
            pl.BlockSpec((1, HIDDEN), lambda i: (0, 0)),
            pl.BlockSpec((1, HIDDEN), lambda i: (0, 0)),
        ],
        out_specs=pl.BlockSpec((BBLK, SEQ, HIDDEN), lambda i: (i, 0, 0)),
        out_shape=jax.ShapeDtypeStruct((B, SEQ, HIDDEN), jnp.float32),
        compiler_params=pltpu.CompilerParams(
            dimension_semantics=("parallel",),
        ),
    )(
        visual_embedding,
        audio_embedding,
        posi_visual_table,
        posi_imu_table,
        modal_table,
        esp_1.reshape(1, HIDDEN),
        esp_2.reshape(1, HIDDEN),
        ln_gamma.reshape(1, HIDDEN),
        ln_beta.reshape(1, HIDDEN),
    )
    return out


# speedup vs baseline: 1.0770x; 1.0770x over previous
import jax
import jax.numpy as jnp
from jax.experimental import pallas as pl
from jax.experimental.pallas import tpu as pltpu

VIS_LEN = 50
IMU_LEN = 200
SEQ = VIS_LEN + IMU_LEN
HIDDEN = 128
EPS = 1e-12
BBLK = 64


def _body(vis_ref, aud_ref, pvt_ref, pit_ref, mt_ref, e1_ref, e2_ref,
          g_ref, b_ref, out_ref):
    m0 = mt_ref[0:1, :]
    m1 = mt_ref[1:2, :]
    bias_vis = pvt_ref[...] + m0
    bias_imu = pit_ref[...] + m1

    vis = vis_ref[...]
    aud = aud_ref[...]
    n = vis.shape[0]

    x0 = jnp.broadcast_to((e1_ref[...] + bias_vis[0:1, :])[None], (n, 1, HIDDEN))
    xv = vis + bias_vis[1:, :][None]
    x1 = jnp.broadcast_to((e2_ref[...] + bias_imu[0:1, :])[None], (n, 1, HIDDEN))
    xa = aud + bias_imu[1:, :][None]
    x = jnp.concatenate([x0, xv, x1, xa], axis=1)

    s1 = jnp.sum(x, axis=-1, keepdims=True)
    s2 = jnp.sum(x * x, axis=-1, keepdims=True)
    mu = s1 * (1.0 / HIDDEN)
    var = s2 * (1.0 / HIDDEN) - mu * mu
    r = jax.lax.rsqrt(var + EPS)
    out_ref[...] = (x - mu) * r * g_ref[...] + b_ref[...]


def kernel(visual_embedding, audio_embedding, posi_visual_table,
           posi_imu_table, modal_table, esp_1, esp_2, ln_gamma, ln_beta):
    B = visual_embedding.shape[0]
    grid = (B // BBLK,)
    out = pl.pallas_call(
        _body,
        grid=grid,
        in_specs=[
            pl.BlockSpec((BBLK, VIS_LEN - 1, HIDDEN), lambda i: (i, 0, 0)),
            pl.BlockSpec((BBLK, IMU_LEN - 1, HIDDEN), lambda i: (i, 0, 0)),
            pl.BlockSpec((VIS_LEN, HIDDEN), lambda i: (0, 0)),
            pl.BlockSpec((IMU_LEN, HIDDEN), lambda i: (0, 0)),
            pl.BlockSpec((2, HIDDEN), lambda i: (0, 0)),
            pl.BlockSpec((1, HIDDEN), lambda i: (0, 0)),
            pl.BlockSpec((1, HIDDEN), lambda i: (0, 0)),
            pl.BlockSpec((1, HIDDEN), lambda i: (0, 0)),
            pl.BlockSpec((1, HIDDEN), lambda i: (0, 0)),
        ],
        out_specs=pl.BlockSpec((BBLK, SEQ, HIDDEN), lambda i: (i, 0, 0)),
        out_shape=jax.ShapeDtypeStruct((B, SEQ, HIDDEN), jnp.float32),
        compiler_params=pltpu.CompilerParams(
            dimension_semantics=("parallel",),
        ),
    )(
        visual_embedding,
        audio_embedding,
        posi_visual_table,
        posi_imu_table,
        modal_table,
        esp_1.reshape(1, HIDDEN),
        esp_2.reshape(1, HIDDEN),
        ln_gamma.reshape(1, HIDDEN),
        ln_beta.reshape(1, HIDDEN),
    )
    return out


# speedup vs baseline: 3.6065x; 3.3487x over previous
import jax
import jax.numpy as jnp
from jax.experimental import pallas as pl
from jax.experimental.pallas import tpu as pltpu

VIS_LEN = 50
IMU_LEN = 200
SEQ = VIS_LEN + IMU_LEN
HIDDEN = 128
EPS = 1e-12
BBLK = 64


def _ln(x, g, b):
    s1 = jnp.sum(x, axis=-1, keepdims=True)
    s2 = jnp.sum(x * x, axis=-1, keepdims=True)
    mu = s1 * (1.0 / HIDDEN)
    var = s2 * (1.0 / HIDDEN) - mu * mu
    r = jax.lax.rsqrt(var + EPS)
    return (x - mu) * r * g + b


def _body(vis_ref, aud_ref, pvt_ref, pit_ref, mt_ref, e1_ref, e2_ref,
          g_ref, b_ref, out_ref):
    m0 = mt_ref[0:1, :]
    m1 = mt_ref[1:2, :]
    bias_vis = pvt_ref[...] + m0
    bias_imu = pit_ref[...] + m1
    g = g_ref[...]
    b = b_ref[...]

    n = out_ref.shape[1]
    y0 = _ln(e1_ref[...] + bias_vis[0:1, :], g, b)
    out_ref[0:1] = jnp.broadcast_to(y0[:, None, :], (1, n, HIDDEN))
    out_ref[1:VIS_LEN] = _ln(vis_ref[...] + bias_vis[1:, None, :],
                             g[None], b[None])
    y1 = _ln(e2_ref[...] + bias_imu[0:1, :], g, b)
    out_ref[VIS_LEN:VIS_LEN + 1] = jnp.broadcast_to(y1[:, None, :], (1, n, HIDDEN))
    out_ref[VIS_LEN + 1:] = _ln(aud_ref[...] + bias_imu[1:, None, :],
                                g[None], b[None])


def kernel(visual_embedding, audio_embedding, posi_visual_table,
           posi_imu_table, modal_table, esp_1, esp_2, ln_gamma, ln_beta):
    B = visual_embedding.shape[0]
    vis_t = jnp.transpose(visual_embedding, (1, 0, 2))
    aud_t = jnp.transpose(audio_embedding, (1, 0, 2))
    grid = (B // BBLK,)
    out_t = pl.pallas_call(
        _body,
        grid=grid,
        in_specs=[
            pl.BlockSpec((VIS_LEN - 1, BBLK, HIDDEN), lambda j: (0, j, 0)),
            pl.BlockSpec((IMU_LEN - 1, BBLK, HIDDEN), lambda j: (0, j, 0)),
            pl.BlockSpec((VIS_LEN, HIDDEN), lambda j: (0, 0)),
            pl.BlockSpec((IMU_LEN, HIDDEN), lambda j: (0, 0)),
            pl.BlockSpec((2, HIDDEN), lambda j: (0, 0)),
            pl.BlockSpec((1, HIDDEN), lambda j: (0, 0)),
            pl.BlockSpec((1, HIDDEN), lambda j: (0, 0)),
            pl.BlockSpec((1, HIDDEN), lambda j: (0, 0)),
            pl.BlockSpec((1, HIDDEN), lambda j: (0, 0)),
        ],
        out_specs=pl.BlockSpec((SEQ, BBLK, HIDDEN), lambda j: (0, j, 0)),
        out_shape=jax.ShapeDtypeStruct((SEQ, B, HIDDEN), jnp.float32),
        compiler_params=pltpu.CompilerParams(
            dimension_semantics=("parallel",),
        ),
    )(
        vis_t,
        aud_t,
        posi_visual_table,
        posi_imu_table,
        modal_table,
        esp_1.reshape(1, HIDDEN),
        esp_2.reshape(1, HIDDEN),
        ln_gamma.reshape(1, HIDDEN),
        ln_beta.reshape(1, HIDDEN),
    )
    return jnp.transpose(out_t, (1, 0, 2))
